# baseline (speedup 1.0000x reference)
import functools

import jax
import jax.numpy as jnp
from jax import lax
from jax.experimental import pallas as pl
from jax.experimental.pallas import tpu as pltpu
from jax.experimental.pallas import tpu_sc as plsc

DIM = 16
HALF = 256
GRP = 16
NG = HALF // GRP


def kernel(table_conf, table_logvar, index_p):
    n_rows = table_conf.shape[0]
    batch = index_p.shape[0]
    info = plsc.get_sparse_core_info()
    nw = info.num_cores * info.num_subcores
    b_per_w = batch // nw

    conf_t = table_conf.T.reshape(2, 8, n_rows)
    logvar_t = table_logvar.T.reshape(2, 8, n_rows)
    idx2 = index_p.reshape(nw, b_per_w)

    mesh = plsc.VectorSubcoreMesh(core_axis_name="c", subcore_axis_name="s")

    @functools.partial(
        pl.kernel,
        mesh=mesh,
        out_type=(
            jax.ShapeDtypeStruct((2, 8, batch), jnp.float32),
            jax.ShapeDtypeStruct((2, 8, batch), jnp.float32),
        ),
        scratch_types=[
            pltpu.VMEM((b_per_w,), jnp.int32),
            pltpu.VMEM((2, 8, 8 * HALF), jnp.float32),
            pltpu.VMEM((2, 8, 8 * HALF), jnp.float32),
            pltpu.VMEM((2, 8, b_per_w), jnp.float32),
            pltpu.VMEM((2, 8, b_per_w), jnp.float32),
            pltpu.SemaphoreType.DMA,
            pltpu.SemaphoreType.DMA,
            pltpu.SemaphoreType.DMA,
            pltpu.SemaphoreType.DMA,
        ],
        compiler_params=pltpu.CompilerParams(needs_layout_passes=False),
    )
    def _gather2(conf_hbm, logvar_hbm, idx_hbm, z_hbm, zl_hbm,
                 idx_v, win_a, win_b, rows_a, rows_b,
                 sem_a0, sem_b0, sem_a1, sem_b1):
        wid = lax.axis_index("s") * info.num_cores + lax.axis_index("c")
        base = pl.multiple_of(wid * b_per_w, 128)
        pltpu.sync_copy(idx_hbm.at[wid], idx_v)
        lane = lax.iota(jnp.int32, 16)

        def issue(p, g, sa, sb):
            vec = idx_v[pl.ds(p * HALF + g * GRP, GRP)]
            for j in range(GRP):
                i = pl.multiple_of(vec[j] & ~7, 8)
                col = g * 128 + j * 8
                pltpu.async_copy(conf_hbm.at[:, :, pl.ds(i, 8)],
                                 win_a.at[:, :, pl.ds(col, 8)], sa)
                pltpu.async_copy(logvar_hbm.at[:, :, pl.ds(i, 8)],
                                 win_b.at[:, :, pl.ds(col, 8)], sb)

        def drain(sa, sb):
            pltpu.make_async_copy(conf_hbm.at[:, :, pl.ds(0, 8 * GRP)],
                                  win_a.at[:, :, pl.ds(0, 8 * GRP)], sa).wait()
            pltpu.make_async_copy(logvar_hbm.at[:, :, pl.ds(0, 8 * GRP)],
                                  win_b.at[:, :, pl.ds(0, 8 * GRP)], sb).wait()

        for p in range(2):
            issue(p, jnp.int32(0), sem_a0, sem_b0)

            def body(g, _):
                nxt = g + 1

                @pl.when((nxt < NG) & (nxt % 2 == 0))
                def _issue_even():
                    issue(p, nxt, sem_a0, sem_b0)

                @pl.when((nxt < NG) & (nxt % 2 == 1))
                def _issue_odd():
                    issue(p, nxt, sem_a1, sem_b1)

                @pl.when(g % 2 == 0)
                def _drain_even():
                    drain(sem_a0, sem_b0)

                @pl.when(g % 2 == 1)
                def _drain_odd():
                    drain(sem_a1, sem_b1)

                vec = idx_v[pl.ds(p * HALF + g * GRP, GRP)]
                pos = g * 128 + lane * 8 + (vec & 7)
                out_c = pl.ds(p * HALF + g * GRP, GRP)
                for t in range(2):
                    for r in range(8):
                        t_vec = jnp.full((16,), t, jnp.int32)
                        r_vec = jnp.full((16,), r, jnp.int32)
                        rows_a[t, r, out_c] = plsc.load_gather(
                            win_a, [t_vec, r_vec, pos])
                        rows_b[t, r, out_c] = plsc.load_gather(
                            win_b, [t_vec, r_vec, pos])
                return _

            lax.fori_loop(0, NG, body, None)

        out_sl = pl.ds(base, b_per_w)
        pltpu.sync_copy(rows_a, z_hbm.at[:, :, out_sl])
        pltpu.sync_copy(rows_b, zl_hbm.at[:, :, out_sl])

    zt, zlt = _gather2(conf_t, logvar_t, idx2)
    z = zt.reshape(DIM, batch).T
    zl = zlt.reshape(DIM, batch).T
    return (z, zl)
